# Initial kernel scaffold; baseline (speedup 1.0000x reference)
import jax
import jax.numpy as jnp
from jax.experimental import pallas as pl


def kernel(x, pos, edge_index, We1, be1, We2, be2, Wx1, bx1, Wx2, bx2, Wh1, bh1, Wh2, bh2, Wfc, bfc):
    raise NotImplementedError("write your pallas kernel here")


# speedup vs baseline: 2.0147x; 2.0147x over previous
import functools

import jax
import jax.numpy as jnp
from jax import lax
from jax.experimental import pallas as pl
from jax.experimental.pallas import tpu as pltpu
from jax.experimental.pallas import tpu_sc as plsc

NC = 2
NS = 16
NW = NC * NS
CHUNK = 128
IG = 8
SB = 4
EB = 512
RB = 1000
SMW = 5


def _silu(v):
    return v * jax.nn.sigmoid(v)


def _tables_body(h_ref, wa_ref, wb_ref, ta_ref, tb_ref):
    h = h_ref[...]
    ta_ref[...] = jnp.dot(h, wa_ref[...], preferred_element_type=jnp.float32)
    tb_ref[...] = jnp.dot(h, wb_ref[...], preferred_element_type=jnp.float32)


def _edge_body(ga_ref, gb_ref, d2_ref, we1c_ref, be1_ref, w2_ref, b2_ref,
               wx1_ref, bx1_ref, wx2_ref, bx2_ref, m_ref, xw_ref):
    a = ga_ref[...]
    b = gb_ref[...]
    we1c = we1c_ref[...]
    outs = []
    for t in range(EB // CHUNK):
        d2row = d2_ref[0, t:t + 1, :]
        outs.append(lax.dot_general(
            d2row, we1c, (((0,), (0,)), ((), ())),
            preferred_element_type=jnp.float32))
    outer = jnp.concatenate(outs, axis=0)
    m1 = a + b + outer + be1_ref[...]
    t_ = _silu(m1)
    m = _silu(jnp.dot(t_, w2_ref[...], preferred_element_type=jnp.float32)
              + b2_ref[...])
    u = _silu(jnp.dot(m, wx1_ref[...], preferred_element_type=jnp.float32)
              + bx1_ref[...])
    m_ref[...] = m
    wx2 = wx2_ref[...]
    rows = []
    for t in range(EB // CHUNK):
        u_t = lax.slice(u, (t * CHUNK, 0), ((t + 1) * CHUNK, 128))
        rows.append(lax.dot_general(
            wx2, u_t, (((0,), (1,)), ((), ())),
            preferred_element_type=jnp.float32))
    xw = jnp.concatenate(rows, axis=0) + bx2_ref[0:1, 0:1]
    xw_ref[...] = xw[None]


def _node_mid_body(h_ref, p8_ref, a0_ref, a1_ref, s0_ref, s1_ref,
                   wh1h_ref, wh1a_ref, bh1_ref, wh2_ref, bh2_ref,
                   wa_ref, wb_ref, hn_ref, p8n_ref, ta_ref, tb_ref):
    h = h_ref[...]
    agg = a0_ref[...] + a1_ref[...]
    s = s0_ref[...] + s1_ref[...]
    pvec = s[:, 0:3]
    sxw = s[:, 3:4]
    cnt = jnp.maximum(s[:, 4:5], 1.0)
    p = p8_ref[...][:, :3]
    p_new = p + (p * sxw - pvec) / cnt
    u = _silu(jnp.dot(h, wh1h_ref[...], preferred_element_type=jnp.float32)
              + jnp.dot(agg, wh1a_ref[...], preferred_element_type=jnp.float32)
              + bh1_ref[...])
    h_new = h + jnp.dot(u, wh2_ref[...], preferred_element_type=jnp.float32) \
        + bh2_ref[...]
    n = h.shape[0]
    p8n = jnp.concatenate([p_new, jnp.zeros((n, 5), jnp.float32)], axis=1)
    hn_ref[...] = h_new
    p8n_ref[...] = p8n
    ta_ref[...] = jnp.dot(h_new, wa_ref[...],
                          preferred_element_type=jnp.float32)
    tb_ref[...] = jnp.dot(h_new, wb_ref[...],
                          preferred_element_type=jnp.float32)


def _node_fin_body(h_ref, a0_ref, a1_ref, wh1h_ref, wh1a_ref, bh1_ref,
                   wh2_ref, bh2_ref, wfc_ref, bfc_ref, y_ref):
    h = h_ref[...]
    agg = a0_ref[...] + a1_ref[...]
    u = _silu(jnp.dot(h, wh1h_ref[...], preferred_element_type=jnp.float32)
              + jnp.dot(agg, wh1a_ref[...], preferred_element_type=jnp.float32)
              + bh1_ref[...])
    h_new = h + jnp.dot(u, wh2_ref[...], preferred_element_type=jnp.float32) \
        + bh2_ref[...]
    y_ref[...] = (jnp.dot(h_new, wfc_ref[...],
                          preferred_element_type=jnp.float32) + bfc_ref[...])


def _wspec(r, c):
    return pl.BlockSpec((r, c), lambda i: (0, 0))


def _tc_tables(n):
    bs = pl.BlockSpec((RB, 128), lambda i: (i, 0))
    return pl.pallas_call(
        _tables_body,
        grid=(n // RB,),
        in_specs=[bs, _wspec(128, 128), _wspec(128, 128)],
        out_specs=[bs, bs],
        out_shape=[jax.ShapeDtypeStruct((n, 128), jnp.float32)] * 2,
    )


def _tc_edge(e_pad):
    bw = pl.BlockSpec((EB, 128), lambda i: (i, 0))
    bx = pl.BlockSpec((1, EB // CHUNK, CHUNK), lambda i: (i, 0, 0))
    return pl.pallas_call(
        _edge_body,
        grid=(e_pad // EB,),
        in_specs=[bw, bw, bx, _wspec(1, 128), _wspec(1, 128),
                  _wspec(128, 128), _wspec(1, 128), _wspec(128, 128),
                  _wspec(1, 128), _wspec(128, 1), _wspec(1, 128)],
        out_specs=[bw, bx],
        out_shape=[jax.ShapeDtypeStruct((e_pad, 128), jnp.float32),
                   jax.ShapeDtypeStruct((e_pad // EB, EB // CHUNK, CHUNK),
                                        jnp.float32)],
    )


def _tc_node_mid(n):
    bh = pl.BlockSpec((RB, 128), lambda i: (i, 0))
    bp = pl.BlockSpec((RB, 8), lambda i: (i, 0))
    bs = pl.BlockSpec((RB, SMW), lambda i: (i, 0))
    return pl.pallas_call(
        _node_mid_body,
        grid=(n // RB,),
        in_specs=[bh, bp, bh, bh, bs, bs, _wspec(128, 128), _wspec(128, 128),
                  _wspec(1, 128), _wspec(128, 128), _wspec(1, 128),
                  _wspec(128, 128), _wspec(128, 128)],
        out_specs=[bh, bp, bh, bh],
        out_shape=[jax.ShapeDtypeStruct((n, 128), jnp.float32),
                   jax.ShapeDtypeStruct((n, 8), jnp.float32),
                   jax.ShapeDtypeStruct((n, 128), jnp.float32),
                   jax.ShapeDtypeStruct((n, 128), jnp.float32)],
    )


def _tc_node_fin(n, co):
    bh = pl.BlockSpec((RB, 128), lambda i: (i, 0))
    return pl.pallas_call(
        _node_fin_body,
        grid=(n // RB,),
        in_specs=[bh, bh, bh, _wspec(128, 128), _wspec(128, 128),
                  _wspec(1, 128), _wspec(128, 128), _wspec(1, 128),
                  _wspec(128, co), _wspec(1, co)],
        out_specs=pl.BlockSpec((RB, co), lambda i: (i, 0)),
        out_shape=jax.ShapeDtypeStruct((n, co), jnp.float32),
    )


def _mesh():
    return plsc.VectorSubcoreMesh(core_axis_name="c", subcore_axis_name="s")


_SC_PARAMS = pltpu.CompilerParams(needs_layout_passes=False)


def _sc_gather(n, e_pad):
    rows = e_pad // CHUNK
    rows_per_w = rows // NW
    groups = rows_per_w // IG
    nblk = e_pad // EB

    @functools.partial(
        pl.kernel,
        out_type=(jax.ShapeDtypeStruct((e_pad, 128), jnp.float32),
                  jax.ShapeDtypeStruct((e_pad, 128), jnp.float32),
                  jax.ShapeDtypeStruct((nblk, EB // CHUNK, CHUNK),
                                       jnp.float32)),
        mesh=_mesh(),
        compiler_params=_SC_PARAMS,
        scratch_types=[pltpu.VMEM((IG, CHUNK), jnp.int32),
                       pltpu.VMEM((IG, CHUNK), jnp.int32),
                       pltpu.VMEM((4 * n,), jnp.float32),
                       pltpu.VMEM((SB * CHUNK, 128), jnp.float32),
                       pltpu.VMEM((IG // (EB // CHUNK), EB // CHUNK, CHUNK),
                                  jnp.float32),
                       pltpu.SemaphoreType.DMA],
    )
    def gk(ta, tb, gr, gc, pf, ga, gb, d2, idxr, idxc, pv, buf, dbuf, sem):
        wid = lax.axis_index("s") * NC + lax.axis_index("c")
        rbase0 = wid * rows_per_w
        pltpu.sync_copy(pf, pv)

        @pl.loop(0, groups)
        def _grp(g):
            rbase = rbase0 + g * IG
            pltpu.sync_copy(gr.at[pl.ds(rbase, IG)], idxr)
            pltpu.sync_copy(gc.at[pl.ds(rbase, IG)], idxc)
            for t in range(IG):
                for q in range(CHUNK // 16):
                    ir = idxr[t, pl.ds(q * 16, 16)] * 4
                    ic = idxc[t, pl.ds(q * 16, 16)] * 4
                    d2v = jnp.zeros((16,), jnp.float32)
                    for k in range(3):
                        dr = plsc.load_gather(pv, [ir + k])
                        dc = plsc.load_gather(pv, [ic + k])
                        dk = dr - dc
                        d2v = d2v + dk * dk
                    dbuf[t // 4, t % 4, pl.ds(q * 16, 16)] = d2v
            pltpu.sync_copy(dbuf, d2.at[pl.ds(rbase // (EB // CHUNK),
                                              IG // (EB // CHUNK))])
            for half in range(IG // SB):
                ebase = (rbase + half * SB) * CHUNK
                descs = [pltpu.async_copy(
                    ta.at[idxr.at[half * SB + t]],
                    buf.at[pl.ds(t * CHUNK, CHUNK)], sem)
                    for t in range(SB)]
                for d in descs:
                    d.wait()
                pltpu.sync_copy(buf, ga.at[pl.ds(ebase, SB * CHUNK)])
                descs = [pltpu.async_copy(
                    tb.at[idxc.at[half * SB + t]],
                    buf.at[pl.ds(t * CHUNK, CHUNK)], sem)
                    for t in range(SB)]
                for d in descs:
                    d.wait()
                pltpu.sync_copy(buf, gb.at[pl.ds(ebase, SB * CHUNK)])

    return gk


def _sc_scatter_m(n, e_pad):
    rows = e_pad // CHUNK
    rows_per_w = rows // NW
    groups = rows_per_w // IG
    SBM = 2

    @functools.partial(
        pl.kernel,
        out_type=jax.ShapeDtypeStruct((NC, n + 1, 128), jnp.float32),
        mesh=_mesh(),
        compiler_params=_SC_PARAMS,
        scratch_types=[pltpu.VMEM((IG, CHUNK), jnp.int32),
                       pltpu.VMEM((SBM * CHUNK, 128), jnp.float32),
                       pltpu.VMEM_SHARED((n + 1, 128), jnp.float32),
                       pltpu.SemaphoreType.DMA],
    )
    def sk(pay, sr, zed, out, idxv, buf, acc, sem):
        c = lax.axis_index("c")
        s = lax.axis_index("s")
        wid = s * NC + c
        rbase0 = wid * rows_per_w

        @pl.when(s == 0)
        def _():
            pltpu.sync_copy(zed, acc)
        plsc.subcore_barrier()

        @pl.loop(0, groups)
        def _grp(g):
            rbase = rbase0 + g * IG
            pltpu.sync_copy(sr.at[pl.ds(rbase, IG)], idxv)
            for half in range(IG // SBM):
                ebase = (rbase + half * SBM) * CHUNK
                pltpu.sync_copy(pay.at[pl.ds(ebase, SBM * CHUNK)], buf)
                for t in range(SBM):
                    pltpu.sync_copy(buf.at[pl.ds(t * CHUNK, CHUNK)],
                                    acc.at[idxv.at[half * SBM + t]], add=True)

        plsc.subcore_barrier()

        @pl.when(s == 0)
        def _():
            pltpu.sync_copy(acc, out.at[c])

    return sk


AROWS = 512


def _sc_scatter_small(n, e_pad):
    rows = e_pad // CHUNK
    rows_per_w = rows // NW
    groups = rows_per_w // IG
    nblk_g = IG // (EB // CHUNK)

    @functools.partial(
        pl.kernel,
        out_type=jax.ShapeDtypeStruct((NC, AROWS, 128), jnp.float32),
        mesh=_mesh(),
        compiler_params=_SC_PARAMS,
        scratch_types=[pltpu.VMEM((IG, CHUNK), jnp.int32),
                       pltpu.VMEM((IG, CHUNK), jnp.int32),
                       pltpu.VMEM((nblk_g, EB // CHUNK, CHUNK), jnp.float32),
                       pltpu.VMEM((4 * n,), jnp.float32),
                       pltpu.VMEM((AROWS, 128), jnp.float32),
                       pltpu.VMEM((AROWS // CHUNK, CHUNK), jnp.int32),
                       pltpu.VMEM_SHARED((AROWS, 128), jnp.float32),
                       pltpu.SemaphoreType.DMA],
    )
    def sk(xw3, gc, sr, pf, zsm, out, idxc, idxr, xwb, pv, accw, idb,
           accs, sem):
        c = lax.axis_index("c")
        s = lax.axis_index("s")
        wid = s * NC + c
        rbase0 = wid * rows_per_w
        pltpu.sync_copy(pf, pv)
        zv = jnp.zeros((16,), jnp.float32)
        iot = lax.iota(jnp.int32, 16)
        for j in range(AROWS // CHUNK):
            for q in range(CHUNK // 16):
                idb[j, pl.ds(q * 16, 16)] = iot + (j * CHUNK + q * 16)

        @pl.loop(0, AROWS)
        def _z(r):
            for q in range(8):
                accw[r, pl.ds(q * 16, 16)] = zv

        @pl.when(s == 0)
        def _():
            pltpu.sync_copy(zsm, accs)
        plsc.subcore_barrier()

        @pl.loop(0, groups)
        def _grp(g):
            rbase = rbase0 + g * IG
            pltpu.sync_copy(gc.at[pl.ds(rbase, IG)], idxc)
            pltpu.sync_copy(sr.at[pl.ds(rbase, IG)], idxr)
            pltpu.sync_copy(xw3.at[pl.ds(rbase // (EB // CHUNK), nblk_g)],
                            xwb)
            ones = jnp.ones((16,), jnp.float32)
            for t in range(IG):
                for q in range(CHUNK // 16):
                    ic = idxc[t, pl.ds(q * 16, 16)] * 4
                    ir = idxr[t, pl.ds(q * 16, 16)] * SMW
                    xw = xwb[t // 4, t % 4, pl.ds(q * 16, 16)]
                    for k in range(3):
                        pc = plsc.load_gather(pv, [ic + k])
                        w = ir + k
                        plsc.addupdate_scatter(
                            accw, [lax.shift_right_logical(w, 7), w & 127],
                            xw * pc)
                    w = ir + 3
                    plsc.addupdate_scatter(
                        accw, [lax.shift_right_logical(w, 7), w & 127], xw)
                    w = ir + 4
                    plsc.addupdate_scatter(
                        accw, [lax.shift_right_logical(w, 7), w & 127], ones)

        for j in range(AROWS // CHUNK):
            pltpu.sync_copy(accw.at[pl.ds(j * CHUNK, CHUNK)],
                            accs.at[idb.at[j]], add=True)
        plsc.subcore_barrier()

        @pl.when(s == 0)
        def _():
            pltpu.sync_copy(accs, out.at[c])

    return sk


def kernel(x, pos, edge_index, We1, be1, We2, be2, Wx1, bx1, Wx2, bx2,
           Wh1, bh1, Wh2, bh2, Wfc, bfc):
    n, h_dim = x.shape
    num_layers = We1.shape[0]
    co = Wfc.shape[1]
    e = edge_index.shape[1]
    step = NW * IG * CHUNK
    e_pad = ((e + step - 1) // step) * step
    padn = e_pad - e

    i32 = jnp.int32
    row = edge_index[0].astype(i32)
    col = edge_index[1].astype(i32)
    g_row = jnp.concatenate([row, jnp.zeros((padn,), i32)]
                            ).reshape(e_pad // CHUNK, CHUNK)
    g_col = jnp.concatenate([col, jnp.zeros((padn,), i32)]
                            ).reshape(e_pad // CHUNK, CHUNK)
    s_row = jnp.concatenate([row, jnp.full((padn,), n, i32)]
                            ).reshape(e_pad // CHUNK, CHUNK)
    p8 = jnp.concatenate([pos, jnp.zeros((n, 5), jnp.float32)], axis=1)
    zeda = jnp.zeros((n + 1, 128), jnp.float32)
    zsm = jnp.zeros((AROWS, 128), jnp.float32)

    tables = _tc_tables(n)
    edge_mlp = _tc_edge(e_pad)
    node_mid = _tc_node_mid(n)
    node_fin = _tc_node_fin(n, co)
    gather = _sc_gather(n, e_pad)
    scatter_m = _sc_scatter_m(n, e_pad)
    scatter_s = _sc_scatter_small(n, e_pad)

    h = x
    ta, tb = tables(h, We1[0, 0:128], We1[0, 128:256])
    for l in range(num_layers):
        p_flat = p8[:, :4].reshape(-1)
        ga, gb, d2 = gather(ta, tb, g_row, g_col, p_flat)
        pay, xw3 = edge_mlp(ga, gb, d2, We1[l, 256:257], be1[l:l + 1],
                            We2[l], be2[l:l + 1], Wx1[l], bx1[l:l + 1],
                            Wx2[l], bx2[l:l + 1])
        accs = scatter_m(pay, s_row, zeda)
        a0 = accs[0, :n]
        a1 = accs[1, :n]
        if l + 1 < num_layers:
            sm = scatter_s(xw3, g_col, s_row, p_flat, zsm)
            sm = sm.reshape(NC, AROWS * 128)[:, :n * SMW].reshape(NC, n, SMW)
            h, p8, ta, tb = node_mid(h, p8, a0, a1, sm[0], sm[1],
                                     Wh1[l, 0:128], Wh1[l, 128:256],
                                     bh1[l:l + 1], Wh2[l], bh2[l:l + 1],
                                     We1[l + 1, 0:128], We1[l + 1, 128:256])
        else:
            y = node_fin(h, a0, a1, Wh1[l, 0:128], Wh1[l, 128:256],
                         bh1[l:l + 1], Wh2[l], bh2[l:l + 1], Wfc,
                         bfc.reshape(1, co))
    return y


# speedup vs baseline: 3.5822x; 1.7781x over previous
import functools

import jax
import jax.numpy as jnp
from jax import lax
from jax.experimental import pallas as pl
from jax.experimental.pallas import tpu as pltpu
from jax.experimental.pallas import tpu_sc as plsc

NC = 2
NS = 16
NW = NC * NS
CHUNK = 128
IG = 8
SB = 4
EB = 512
RB = 1000
SMW = 5


def _silu(v):
    return v * jax.nn.sigmoid(v)


def _tables_body(h_ref, wa_ref, wb_ref, ta_ref, tb_ref):
    h = h_ref[...]
    ta_ref[...] = jnp.dot(h, wa_ref[...], preferred_element_type=jnp.float32)
    tb_ref[...] = jnp.dot(h, wb_ref[...], preferred_element_type=jnp.float32)


def _edge_body(ga_ref, gb_ref, d2_ref, we1c_ref, be1_ref, w2_ref, b2_ref,
               wx1_ref, bx1_ref, wx2_ref, bx2_ref, m_ref, xw_ref):
    a = ga_ref[...]
    b = gb_ref[...]
    we1c = we1c_ref[...]
    outs = []
    for t in range(EB // CHUNK):
        d2row = d2_ref[0, t:t + 1, :]
        outs.append(lax.dot_general(
            d2row, we1c, (((0,), (0,)), ((), ())),
            preferred_element_type=jnp.float32))
    outer = jnp.concatenate(outs, axis=0)
    m1 = a + b + outer + be1_ref[...]
    t_ = _silu(m1)
    m = _silu(jnp.dot(t_, w2_ref[...], preferred_element_type=jnp.float32)
              + b2_ref[...])
    u = _silu(jnp.dot(m, wx1_ref[...], preferred_element_type=jnp.float32)
              + bx1_ref[...])
    m_ref[...] = m
    wx2 = wx2_ref[...]
    rows = []
    for t in range(EB // CHUNK):
        u_t = lax.slice(u, (t * CHUNK, 0), ((t + 1) * CHUNK, 128))
        rows.append(lax.dot_general(
            wx2, u_t, (((0,), (1,)), ((), ())),
            preferred_element_type=jnp.float32))
    xw = jnp.concatenate(rows, axis=0) + bx2_ref[0:1, 0:1]
    xw_ref[...] = xw[None]


def _node_mid_body(h_ref, p8_ref, a0_ref, a1_ref, s0_ref, s1_ref,
                   wh1h_ref, wh1a_ref, bh1_ref, wh2_ref, bh2_ref,
                   wa_ref, wb_ref, hn_ref, p8n_ref, ta_ref, tb_ref):
    h = h_ref[...]
    agg = a0_ref[...] + a1_ref[...]
    s = s0_ref[...] + s1_ref[...]
    pvec = s[:, 0:3]
    sxw = s[:, 3:4]
    cnt = jnp.maximum(s[:, 4:5], 1.0)
    p = p8_ref[...][:, :3]
    p_new = p + (p * sxw - pvec) / cnt
    u = _silu(jnp.dot(h, wh1h_ref[...], preferred_element_type=jnp.float32)
              + jnp.dot(agg, wh1a_ref[...], preferred_element_type=jnp.float32)
              + bh1_ref[...])
    h_new = h + jnp.dot(u, wh2_ref[...], preferred_element_type=jnp.float32) \
        + bh2_ref[...]
    n = h.shape[0]
    p8n = jnp.concatenate([p_new, jnp.zeros((n, 5), jnp.float32)], axis=1)
    hn_ref[...] = h_new
    p8n_ref[...] = p8n
    ta_ref[...] = jnp.dot(h_new, wa_ref[...],
                          preferred_element_type=jnp.float32)
    tb_ref[...] = jnp.dot(h_new, wb_ref[...],
                          preferred_element_type=jnp.float32)


def _node_fin_body(h_ref, a0_ref, a1_ref, wh1h_ref, wh1a_ref, bh1_ref,
                   wh2_ref, bh2_ref, wfc_ref, bfc_ref, y_ref):
    h = h_ref[...]
    agg = a0_ref[...] + a1_ref[...]
    u = _silu(jnp.dot(h, wh1h_ref[...], preferred_element_type=jnp.float32)
              + jnp.dot(agg, wh1a_ref[...], preferred_element_type=jnp.float32)
              + bh1_ref[...])
    h_new = h + jnp.dot(u, wh2_ref[...], preferred_element_type=jnp.float32) \
        + bh2_ref[...]
    y_ref[...] = (jnp.dot(h_new, wfc_ref[...],
                          preferred_element_type=jnp.float32) + bfc_ref[...])


def _wspec(r, c):
    return pl.BlockSpec((r, c), lambda i: (0, 0))


def _tc_tables(n):
    bs = pl.BlockSpec((RB, 128), lambda i: (i, 0))
    return pl.pallas_call(
        _tables_body,
        grid=(n // RB,),
        in_specs=[bs, _wspec(128, 128), _wspec(128, 128)],
        out_specs=[bs, bs],
        out_shape=[jax.ShapeDtypeStruct((n, 128), jnp.float32)] * 2,
    )


def _tc_edge(e_pad):
    bw = pl.BlockSpec((EB, 128), lambda i: (i, 0))
    bx = pl.BlockSpec((1, EB // CHUNK, CHUNK), lambda i: (i, 0, 0))
    return pl.pallas_call(
        _edge_body,
        grid=(e_pad // EB,),
        in_specs=[bw, bw, bx, _wspec(1, 128), _wspec(1, 128),
                  _wspec(128, 128), _wspec(1, 128), _wspec(128, 128),
                  _wspec(1, 128), _wspec(128, 1), _wspec(1, 128)],
        out_specs=[bw, bx],
        out_shape=[jax.ShapeDtypeStruct((e_pad, 128), jnp.float32),
                   jax.ShapeDtypeStruct((e_pad // EB, EB // CHUNK, CHUNK),
                                        jnp.float32)],
    )


def _tc_node_mid(n):
    bh = pl.BlockSpec((RB, 128), lambda i: (i, 0))
    bp = pl.BlockSpec((RB, 8), lambda i: (i, 0))
    bs = pl.BlockSpec((RB, SMW), lambda i: (i, 0))
    return pl.pallas_call(
        _node_mid_body,
        grid=(n // RB,),
        in_specs=[bh, bp, bh, bh, bs, bs, _wspec(128, 128), _wspec(128, 128),
                  _wspec(1, 128), _wspec(128, 128), _wspec(1, 128),
                  _wspec(128, 128), _wspec(128, 128)],
        out_specs=[bh, bp, bh, bh],
        out_shape=[jax.ShapeDtypeStruct((n, 128), jnp.float32),
                   jax.ShapeDtypeStruct((n, 8), jnp.float32),
                   jax.ShapeDtypeStruct((n, 128), jnp.float32),
                   jax.ShapeDtypeStruct((n, 128), jnp.float32)],
    )


def _tc_node_fin(n, co):
    bh = pl.BlockSpec((RB, 128), lambda i: (i, 0))
    return pl.pallas_call(
        _node_fin_body,
        grid=(n // RB,),
        in_specs=[bh, bh, bh, _wspec(128, 128), _wspec(128, 128),
                  _wspec(1, 128), _wspec(128, 128), _wspec(1, 128),
                  _wspec(128, co), _wspec(1, co)],
        out_specs=pl.BlockSpec((RB, co), lambda i: (i, 0)),
        out_shape=jax.ShapeDtypeStruct((n, co), jnp.float32),
    )


def _mesh():
    return plsc.VectorSubcoreMesh(core_axis_name="c", subcore_axis_name="s")


_SC_PARAMS = pltpu.CompilerParams(needs_layout_passes=False)


def _sc_dist2(n, e_pad):
    rows = e_pad // CHUNK
    rows_per_w = rows // NW
    d_groups = rows_per_w // IG
    nblk = e_pad // EB

    @functools.partial(
        pl.kernel,
        out_type=jax.ShapeDtypeStruct((nblk, EB // CHUNK, CHUNK),
                                      jnp.float32),
        mesh=_mesh(),
        compiler_params=_SC_PARAMS,
        scratch_types=[pltpu.VMEM((IG, CHUNK), jnp.int32),
                       pltpu.VMEM((IG, CHUNK), jnp.int32),
                       pltpu.VMEM((4 * n,), jnp.float32),
                       pltpu.VMEM((IG // (EB // CHUNK), EB // CHUNK, CHUNK),
                                  jnp.float32)],
    )
    def dk(gr, gc, pf, d2, idxr, idxc, pv, dbuf):
        wid = lax.axis_index("s") * NC + lax.axis_index("c")
        rbase0 = wid * rows_per_w
        pltpu.sync_copy(pf, pv)

        @pl.loop(0, d_groups)
        def _dgrp(g):
            rbase = rbase0 + g * IG
            pltpu.sync_copy(gr.at[pl.ds(rbase, IG)], idxr)
            pltpu.sync_copy(gc.at[pl.ds(rbase, IG)], idxc)
            for t in range(IG):
                for q in range(CHUNK // 16):
                    ir = idxr[t, pl.ds(q * 16, 16)] * 4
                    ic = idxc[t, pl.ds(q * 16, 16)] * 4
                    d2v = jnp.zeros((16,), jnp.float32)
                    for k in range(3):
                        dr = plsc.load_gather(pv, [ir + k])
                        dc = plsc.load_gather(pv, [ic + k])
                        dk_ = dr - dc
                        d2v = d2v + dk_ * dk_
                    dbuf[t // 4, t % 4, pl.ds(q * 16, 16)] = d2v
            pltpu.sync_copy(dbuf, d2.at[pl.ds(rbase // (EB // CHUNK),
                                              IG // (EB // CHUNK))])

    return dk


GSB = 2


def _sc_gather(n, e_pad):
    rows = e_pad // CHUNK
    rows_per_s = rows // NS
    g_groups = rows_per_s // IG

    @functools.partial(
        pl.kernel,
        out_type=(jax.ShapeDtypeStruct((e_pad, 128), jnp.float32),
                  jax.ShapeDtypeStruct((e_pad, 128), jnp.float32)),
        mesh=_mesh(),
        compiler_params=_SC_PARAMS,
        scratch_types=[pltpu.VMEM((IG, CHUNK), jnp.int32),
                       pltpu.VMEM((GSB * CHUNK, 128), jnp.float32),
                       pltpu.VMEM_SHARED((n, 128), jnp.float32),
                       pltpu.SemaphoreType.DMA],
    )
    def gk(ta, tb, gr, gc, ga, gb, idx, buf, tab, sem):
        c = lax.axis_index("c")
        s = lax.axis_index("s")

        @pl.when(jnp.logical_and(s == 0, c == 0))
        def _():
            pltpu.sync_copy(ta, tab)

        @pl.when(jnp.logical_and(s == 0, c == 1))
        def _():
            pltpu.sync_copy(tb, tab)
        plsc.subcore_barrier()

        gbase0 = s * rows_per_s

        @pl.loop(0, g_groups)
        def _ggrp(g):
            rbase = gbase0 + g * IG

            @pl.when(c == 0)
            def _():
                pltpu.sync_copy(gr.at[pl.ds(rbase, IG)], idx)

            @pl.when(c == 1)
            def _():
                pltpu.sync_copy(gc.at[pl.ds(rbase, IG)], idx)
            for half in range(IG // GSB):
                ebase = (rbase + half * GSB) * CHUNK
                descs = [pltpu.async_copy(
                    tab.at[idx.at[half * GSB + t]],
                    buf.at[pl.ds(t * CHUNK, CHUNK)], sem)
                    for t in range(GSB)]
                for d in descs:
                    d.wait()

                @pl.when(c == 0)
                def _():
                    pltpu.sync_copy(buf, ga.at[pl.ds(ebase, GSB * CHUNK)])

                @pl.when(c == 1)
                def _():
                    pltpu.sync_copy(buf, gb.at[pl.ds(ebase, GSB * CHUNK)])

    return gk


def _sc_scatter_m(n, e_pad):
    rows = e_pad // CHUNK
    rows_per_w = rows // NW
    groups = rows_per_w // IG
    SBM = 2

    @functools.partial(
        pl.kernel,
        out_type=jax.ShapeDtypeStruct((NC, n + 1, 128), jnp.float32),
        mesh=_mesh(),
        compiler_params=_SC_PARAMS,
        scratch_types=[pltpu.VMEM((IG, CHUNK), jnp.int32),
                       pltpu.VMEM((SBM * CHUNK, 128), jnp.float32),
                       pltpu.VMEM_SHARED((n + 1, 128), jnp.float32),
                       pltpu.SemaphoreType.DMA],
    )
    def sk(pay, sr, zed, out, idxv, buf, acc, sem):
        c = lax.axis_index("c")
        s = lax.axis_index("s")
        wid = s * NC + c
        rbase0 = wid * rows_per_w

        @pl.when(s == 0)
        def _():
            pltpu.sync_copy(zed, acc)
        plsc.subcore_barrier()

        @pl.loop(0, groups)
        def _grp(g):
            rbase = rbase0 + g * IG
            pltpu.sync_copy(sr.at[pl.ds(rbase, IG)], idxv)
            for half in range(IG // SBM):
                ebase = (rbase + half * SBM) * CHUNK
                pltpu.sync_copy(pay.at[pl.ds(ebase, SBM * CHUNK)], buf)
                for t in range(SBM):
                    pltpu.sync_copy(buf.at[pl.ds(t * CHUNK, CHUNK)],
                                    acc.at[idxv.at[half * SBM + t]], add=True)

        plsc.subcore_barrier()

        @pl.when(s == 0)
        def _():
            pltpu.sync_copy(acc, out.at[c])

    return sk


AROWS = 512


def _sc_scatter_small(n, e_pad):
    rows = e_pad // CHUNK
    rows_per_w = rows // NW
    groups = rows_per_w // IG
    nblk_g = IG // (EB // CHUNK)

    @functools.partial(
        pl.kernel,
        out_type=jax.ShapeDtypeStruct((NC, AROWS, 128), jnp.float32),
        mesh=_mesh(),
        compiler_params=_SC_PARAMS,
        scratch_types=[pltpu.VMEM((IG, CHUNK), jnp.int32),
                       pltpu.VMEM((IG, CHUNK), jnp.int32),
                       pltpu.VMEM((nblk_g, EB // CHUNK, CHUNK), jnp.float32),
                       pltpu.VMEM((4 * n,), jnp.float32),
                       pltpu.VMEM((AROWS, 128), jnp.float32),
                       pltpu.VMEM((AROWS // CHUNK, CHUNK), jnp.int32),
                       pltpu.VMEM_SHARED((AROWS, 128), jnp.float32),
                       pltpu.SemaphoreType.DMA],
    )
    def sk(xw3, gc, sr, pf, zsm, out, idxc, idxr, xwb, pv, accw, idb,
           accs, sem):
        c = lax.axis_index("c")
        s = lax.axis_index("s")
        wid = s * NC + c
        rbase0 = wid * rows_per_w
        pltpu.sync_copy(pf, pv)
        zv = jnp.zeros((16,), jnp.float32)
        iot = lax.iota(jnp.int32, 16)
        for j in range(AROWS // CHUNK):
            for q in range(CHUNK // 16):
                idb[j, pl.ds(q * 16, 16)] = iot + (j * CHUNK + q * 16)

        @pl.loop(0, AROWS)
        def _z(r):
            for q in range(8):
                accw[r, pl.ds(q * 16, 16)] = zv

        @pl.when(s == 0)
        def _():
            pltpu.sync_copy(zsm, accs)
        plsc.subcore_barrier()

        @pl.loop(0, groups)
        def _grp(g):
            rbase = rbase0 + g * IG
            pltpu.sync_copy(gc.at[pl.ds(rbase, IG)], idxc)
            pltpu.sync_copy(sr.at[pl.ds(rbase, IG)], idxr)
            pltpu.sync_copy(xw3.at[pl.ds(rbase // (EB // CHUNK), nblk_g)],
                            xwb)
            ones = jnp.ones((16,), jnp.float32)
            for t in range(IG):
                for q in range(CHUNK // 16):
                    ic = idxc[t, pl.ds(q * 16, 16)] * 4
                    ir = idxr[t, pl.ds(q * 16, 16)] * SMW
                    xw = xwb[t // 4, t % 4, pl.ds(q * 16, 16)]
                    for k in range(3):
                        pc = plsc.load_gather(pv, [ic + k])
                        w = ir + k
                        plsc.addupdate_scatter(
                            accw, [lax.shift_right_logical(w, 7), w & 127],
                            xw * pc)
                    w = ir + 3
                    plsc.addupdate_scatter(
                        accw, [lax.shift_right_logical(w, 7), w & 127], xw)
                    w = ir + 4
                    plsc.addupdate_scatter(
                        accw, [lax.shift_right_logical(w, 7), w & 127], ones)

        for j in range(AROWS // CHUNK):
            pltpu.sync_copy(accw.at[pl.ds(j * CHUNK, CHUNK)],
                            accs.at[idb.at[j]], add=True)
        plsc.subcore_barrier()

        @pl.when(s == 0)
        def _():
            pltpu.sync_copy(accs, out.at[c])

    return sk


def kernel(x, pos, edge_index, We1, be1, We2, be2, Wx1, bx1, Wx2, bx2,
           Wh1, bh1, Wh2, bh2, Wfc, bfc):
    n, h_dim = x.shape
    num_layers = We1.shape[0]
    co = Wfc.shape[1]
    e = edge_index.shape[1]
    step = NW * IG * CHUNK
    e_pad = ((e + step - 1) // step) * step
    padn = e_pad - e

    i32 = jnp.int32
    row = edge_index[0].astype(i32)
    col = edge_index[1].astype(i32)
    g_row = jnp.concatenate([row, jnp.zeros((padn,), i32)]
                            ).reshape(e_pad // CHUNK, CHUNK)
    g_col = jnp.concatenate([col, jnp.zeros((padn,), i32)]
                            ).reshape(e_pad // CHUNK, CHUNK)
    s_row = jnp.concatenate([row, jnp.full((padn,), n, i32)]
                            ).reshape(e_pad // CHUNK, CHUNK)
    p8 = jnp.concatenate([pos, jnp.zeros((n, 5), jnp.float32)], axis=1)
    zeda = jnp.zeros((n + 1, 128), jnp.float32)
    zsm = jnp.zeros((AROWS, 128), jnp.float32)

    tables = _tc_tables(n)
    edge_mlp = _tc_edge(e_pad)
    node_mid = _tc_node_mid(n)
    node_fin = _tc_node_fin(n, co)
    dist2 = _sc_dist2(n, e_pad)
    gather = _sc_gather(n, e_pad)
    scatter_m = _sc_scatter_m(n, e_pad)
    scatter_s = _sc_scatter_small(n, e_pad)

    h = x
    ta, tb = tables(h, We1[0, 0:128], We1[0, 128:256])
    for l in range(num_layers):
        p_flat = p8[:, :4].reshape(-1)
        d2 = dist2(g_row, g_col, p_flat)
        ga, gb = gather(ta, tb, g_row, g_col)
        pay, xw3 = edge_mlp(ga, gb, d2, We1[l, 256:257], be1[l:l + 1],
                            We2[l], be2[l:l + 1], Wx1[l], bx1[l:l + 1],
                            Wx2[l], bx2[l:l + 1])
        accs = scatter_m(pay, s_row, zeda)
        a0 = accs[0, :n]
        a1 = accs[1, :n]
        if l + 1 < num_layers:
            sm = scatter_s(xw3, g_col, s_row, p_flat, zsm)
            sm = sm.reshape(NC, AROWS * 128)[:, :n * SMW].reshape(NC, n, SMW)
            h, p8, ta, tb = node_mid(h, p8, a0, a1, sm[0], sm[1],
                                     Wh1[l, 0:128], Wh1[l, 128:256],
                                     bh1[l:l + 1], Wh2[l], bh2[l:l + 1],
                                     We1[l + 1, 0:128], We1[l + 1, 128:256])
        else:
            y = node_fin(h, a0, a1, Wh1[l, 0:128], Wh1[l, 128:256],
                         bh1[l:l + 1], Wh2[l], bh2[l:l + 1], Wfc,
                         bfc.reshape(1, co))
    return y


# speedup vs baseline: 4.1600x; 1.1613x over previous
import functools

import jax
import jax.numpy as jnp
from jax import lax
from jax.experimental import pallas as pl
from jax.experimental.pallas import tpu as pltpu
from jax.experimental.pallas import tpu_sc as plsc

NC = 2
NS = 16
NW = NC * NS
CHUNK = 128
IG = 8
SB = 4
EB = 512
RB = 1000
SMW = 5


def _silu(v):
    return v * jax.nn.sigmoid(v)


def _tables_body(h_ref, wa_ref, wb_ref, ta_ref, tb_ref):
    h = h_ref[...]
    ta_ref[...] = jnp.dot(h, wa_ref[...], preferred_element_type=jnp.float32)
    tb_ref[...] = jnp.dot(h, wb_ref[...], preferred_element_type=jnp.float32)


def _edge_body(ga_ref, gb_ref, d2_ref, we1c_ref, be1_ref, w2_ref, b2_ref,
               wx1_ref, bx1_ref, wx2_ref, bx2_ref, m_ref, xw_ref):
    a = ga_ref[...]
    b = gb_ref[...]
    we1c = we1c_ref[...]
    outs = []
    for t in range(EB // CHUNK):
        d2row = d2_ref[0, t:t + 1, :]
        outs.append(lax.dot_general(
            d2row, we1c, (((0,), (0,)), ((), ())),
            preferred_element_type=jnp.float32))
    outer = jnp.concatenate(outs, axis=0)
    m1 = a + b + outer + be1_ref[...]
    t_ = _silu(m1)
    m = _silu(jnp.dot(t_, w2_ref[...], preferred_element_type=jnp.float32)
              + b2_ref[...])
    u = _silu(jnp.dot(m, wx1_ref[...], preferred_element_type=jnp.float32)
              + bx1_ref[...])
    m_ref[...] = m
    wx2 = wx2_ref[...]
    rows = []
    for t in range(EB // CHUNK):
        u_t = lax.slice(u, (t * CHUNK, 0), ((t + 1) * CHUNK, 128))
        rows.append(lax.dot_general(
            wx2, u_t, (((0,), (1,)), ((), ())),
            preferred_element_type=jnp.float32))
    xw = jnp.concatenate(rows, axis=0) + bx2_ref[0:1, 0:1]
    xw_ref[...] = xw[None]


def _node_mid_body(h_ref, p8_ref, a0_ref, a1_ref, s0_ref, s1_ref,
                   wh1h_ref, wh1a_ref, bh1_ref, wh2_ref, bh2_ref,
                   wa_ref, wb_ref, hn_ref, p8n_ref, ta_ref, tb_ref):
    h = h_ref[...]
    agg = a0_ref[...] + a1_ref[...]
    s = s0_ref[...] + s1_ref[...]
    pvec = s[:, 0:3]
    sxw = s[:, 3:4]
    cnt = jnp.maximum(s[:, 4:5], 1.0)
    p = p8_ref[...][:, :3]
    p_new = p + (p * sxw - pvec) / cnt
    u = _silu(jnp.dot(h, wh1h_ref[...], preferred_element_type=jnp.float32)
              + jnp.dot(agg, wh1a_ref[...], preferred_element_type=jnp.float32)
              + bh1_ref[...])
    h_new = h + jnp.dot(u, wh2_ref[...], preferred_element_type=jnp.float32) \
        + bh2_ref[...]
    n = h.shape[0]
    p8n = jnp.concatenate([p_new, jnp.zeros((n, 5), jnp.float32)], axis=1)
    hn_ref[...] = h_new
    p8n_ref[...] = p8n
    ta_ref[...] = jnp.dot(h_new, wa_ref[...],
                          preferred_element_type=jnp.float32)
    tb_ref[...] = jnp.dot(h_new, wb_ref[...],
                          preferred_element_type=jnp.float32)


def _node_fin_body(h_ref, a0_ref, a1_ref, wh1h_ref, wh1a_ref, bh1_ref,
                   wh2_ref, bh2_ref, wfc_ref, bfc_ref, y_ref):
    h = h_ref[...]
    agg = a0_ref[...] + a1_ref[...]
    u = _silu(jnp.dot(h, wh1h_ref[...], preferred_element_type=jnp.float32)
              + jnp.dot(agg, wh1a_ref[...], preferred_element_type=jnp.float32)
              + bh1_ref[...])
    h_new = h + jnp.dot(u, wh2_ref[...], preferred_element_type=jnp.float32) \
        + bh2_ref[...]
    y_ref[...] = (jnp.dot(h_new, wfc_ref[...],
                          preferred_element_type=jnp.float32) + bfc_ref[...])


def _wspec(r, c):
    return pl.BlockSpec((r, c), lambda i: (0, 0))


def _tc_tables(n):
    bs = pl.BlockSpec((RB, 128), lambda i: (i, 0))
    return pl.pallas_call(
        _tables_body,
        grid=(n // RB,),
        in_specs=[bs, _wspec(128, 128), _wspec(128, 128)],
        out_specs=[bs, bs],
        out_shape=[jax.ShapeDtypeStruct((n, 128), jnp.float32)] * 2,
    )


def _tc_edge(e_pad):
    bw = pl.BlockSpec((EB, 128), lambda i: (i, 0))
    bx = pl.BlockSpec((1, EB // CHUNK, CHUNK), lambda i: (i, 0, 0))
    return pl.pallas_call(
        _edge_body,
        grid=(e_pad // EB,),
        in_specs=[bw, bw, bx, _wspec(1, 128), _wspec(1, 128),
                  _wspec(128, 128), _wspec(1, 128), _wspec(128, 128),
                  _wspec(1, 128), _wspec(128, 1), _wspec(1, 128)],
        out_specs=[bw, bx],
        out_shape=[jax.ShapeDtypeStruct((e_pad, 128), jnp.float32),
                   jax.ShapeDtypeStruct((e_pad // EB, EB // CHUNK, CHUNK),
                                        jnp.float32)],
    )


def _tc_node_mid(n):
    bh = pl.BlockSpec((RB, 128), lambda i: (i, 0))
    bp = pl.BlockSpec((RB, 8), lambda i: (i, 0))
    bs = pl.BlockSpec((RB, SMW), lambda i: (i, 0))
    return pl.pallas_call(
        _node_mid_body,
        grid=(n // RB,),
        in_specs=[bh, bp, bh, bh, bs, bs, _wspec(128, 128), _wspec(128, 128),
                  _wspec(1, 128), _wspec(128, 128), _wspec(1, 128),
                  _wspec(128, 128), _wspec(128, 128)],
        out_specs=[bh, bp, bh, bh],
        out_shape=[jax.ShapeDtypeStruct((n, 128), jnp.float32),
                   jax.ShapeDtypeStruct((n, 8), jnp.float32),
                   jax.ShapeDtypeStruct((n, 128), jnp.float32),
                   jax.ShapeDtypeStruct((n, 128), jnp.float32)],
    )


def _tc_node_fin(n, co):
    bh = pl.BlockSpec((RB, 128), lambda i: (i, 0))
    return pl.pallas_call(
        _node_fin_body,
        grid=(n // RB,),
        in_specs=[bh, bh, bh, _wspec(128, 128), _wspec(128, 128),
                  _wspec(1, 128), _wspec(128, 128), _wspec(1, 128),
                  _wspec(128, co), _wspec(1, co)],
        out_specs=pl.BlockSpec((RB, co), lambda i: (i, 0)),
        out_shape=jax.ShapeDtypeStruct((n, co), jnp.float32),
    )


def _mesh():
    return plsc.VectorSubcoreMesh(core_axis_name="c", subcore_axis_name="s")


_SC_PARAMS = pltpu.CompilerParams(needs_layout_passes=False)


def _sc_dist2(n, e_pad):
    rows = e_pad // CHUNK
    rows_per_w = rows // NW
    d_groups = rows_per_w // IG
    nblk = e_pad // EB

    @functools.partial(
        pl.kernel,
        out_type=jax.ShapeDtypeStruct((nblk, EB // CHUNK, CHUNK),
                                      jnp.float32),
        mesh=_mesh(),
        compiler_params=_SC_PARAMS,
        scratch_types=[pltpu.VMEM((IG, CHUNK), jnp.int32),
                       pltpu.VMEM((IG, CHUNK), jnp.int32),
                       pltpu.VMEM((4 * n,), jnp.float32),
                       pltpu.VMEM((IG // (EB // CHUNK), EB // CHUNK, CHUNK),
                                  jnp.float32)],
    )
    def dk(gr, gc, pf, d2, idxr, idxc, pv, dbuf):
        wid = lax.axis_index("s") * NC + lax.axis_index("c")
        rbase0 = wid * rows_per_w
        pltpu.sync_copy(pf, pv)

        @pl.loop(0, d_groups)
        def _dgrp(g):
            rbase = rbase0 + g * IG
            pltpu.sync_copy(gr.at[pl.ds(rbase, IG)], idxr)
            pltpu.sync_copy(gc.at[pl.ds(rbase, IG)], idxc)
            for t in range(IG):
                for q in range(CHUNK // 16):
                    ir = idxr[t, pl.ds(q * 16, 16)] * 4
                    ic = idxc[t, pl.ds(q * 16, 16)] * 4
                    d2v = jnp.zeros((16,), jnp.float32)
                    for k in range(3):
                        dr = plsc.load_gather(pv, [ir + k])
                        dc = plsc.load_gather(pv, [ic + k])
                        dk_ = dr - dc
                        d2v = d2v + dk_ * dk_
                    dbuf[t // 4, t % 4, pl.ds(q * 16, 16)] = d2v
            pltpu.sync_copy(dbuf, d2.at[pl.ds(rbase // (EB // CHUNK),
                                              IG // (EB // CHUNK))])

    return dk


GSB = 2


def _sc_gather(n, e_pad):
    rows = e_pad // CHUNK
    rows_per_s = rows // NS
    g_groups = rows_per_s // IG

    @functools.partial(
        pl.kernel,
        out_type=(jax.ShapeDtypeStruct((e_pad, 128), jnp.float32),
                  jax.ShapeDtypeStruct((e_pad, 128), jnp.float32)),
        mesh=_mesh(),
        compiler_params=_SC_PARAMS,
        scratch_types=[pltpu.VMEM((IG, CHUNK), jnp.int32),
                       pltpu.VMEM((GSB * CHUNK, 128), jnp.float32),
                       pltpu.VMEM_SHARED((n, 128), jnp.float32),
                       pltpu.SemaphoreType.DMA],
    )
    def gk(ta, tb, gr, gc, ga, gb, idx, buf, tab, sem):
        c = lax.axis_index("c")
        s = lax.axis_index("s")

        @pl.when(jnp.logical_and(s == 0, c == 0))
        def _():
            pltpu.sync_copy(ta, tab)

        @pl.when(jnp.logical_and(s == 0, c == 1))
        def _():
            pltpu.sync_copy(tb, tab)
        plsc.subcore_barrier()

        gbase0 = s * rows_per_s

        @pl.loop(0, g_groups)
        def _ggrp(g):
            rbase = gbase0 + g * IG

            @pl.when(c == 0)
            def _():
                pltpu.sync_copy(gr.at[pl.ds(rbase, IG)], idx)

            @pl.when(c == 1)
            def _():
                pltpu.sync_copy(gc.at[pl.ds(rbase, IG)], idx)
            for half in range(IG // GSB):
                ebase = (rbase + half * GSB) * CHUNK
                descs = [pltpu.async_copy(
                    tab.at[idx.at[half * GSB + t]],
                    buf.at[pl.ds(t * CHUNK, CHUNK)], sem)
                    for t in range(GSB)]
                for d in descs:
                    d.wait()

                @pl.when(c == 0)
                def _():
                    pltpu.sync_copy(buf, ga.at[pl.ds(ebase, GSB * CHUNK)])

                @pl.when(c == 1)
                def _():
                    pltpu.sync_copy(buf, gb.at[pl.ds(ebase, GSB * CHUNK)])

    return gk


def _sc_scatter_m(n, e_pad):
    rows = e_pad // CHUNK
    rows_per_w = rows // NW
    groups = rows_per_w // IG
    SBM = 2

    @functools.partial(
        pl.kernel,
        out_type=jax.ShapeDtypeStruct((NC, n + 1, 128), jnp.float32),
        mesh=_mesh(),
        compiler_params=_SC_PARAMS,
        scratch_types=[pltpu.VMEM((IG, CHUNK), jnp.int32),
                       pltpu.VMEM((SBM * CHUNK, 128), jnp.float32),
                       pltpu.VMEM_SHARED((n + 1, 128), jnp.float32),
                       pltpu.SemaphoreType.DMA],
    )
    def sk(pay, sr, zed, out, idxv, buf, acc, sem):
        c = lax.axis_index("c")
        s = lax.axis_index("s")
        wid = s * NC + c
        rbase0 = wid * rows_per_w

        @pl.when(s == 0)
        def _():
            pltpu.sync_copy(zed.at[c], acc)
        plsc.subcore_barrier()

        @pl.loop(0, groups)
        def _grp(g):
            rbase = rbase0 + g * IG
            pltpu.sync_copy(sr.at[pl.ds(rbase, IG)], idxv)
            for half in range(IG // SBM):
                ebase = (rbase + half * SBM) * CHUNK
                pltpu.sync_copy(pay.at[pl.ds(ebase, SBM * CHUNK)], buf)
                for t in range(SBM):
                    pltpu.sync_copy(buf.at[pl.ds(t * CHUNK, CHUNK)],
                                    acc.at[idxv.at[half * SBM + t]], add=True)

        plsc.subcore_barrier()

        @pl.when(s == 0)
        def _():
            pltpu.sync_copy(acc, out.at[c])

    return sk


AROWS = 512


def _sc_scatter_small(n, e_pad):
    rows = e_pad // CHUNK
    rows_per_w = rows // NW
    groups = rows_per_w // IG
    nblk_g = IG // (EB // CHUNK)

    @functools.partial(
        pl.kernel,
        out_type=jax.ShapeDtypeStruct((NC, AROWS, 128), jnp.float32),
        mesh=_mesh(),
        compiler_params=_SC_PARAMS,
        scratch_types=[pltpu.VMEM((IG, CHUNK), jnp.int32),
                       pltpu.VMEM((IG, CHUNK), jnp.int32),
                       pltpu.VMEM((nblk_g, EB // CHUNK, CHUNK), jnp.float32),
                       pltpu.VMEM((4 * n,), jnp.float32),
                       pltpu.VMEM((AROWS, 128), jnp.float32),
                       pltpu.VMEM((AROWS // CHUNK, CHUNK), jnp.int32),
                       pltpu.VMEM_SHARED((AROWS, 128), jnp.float32),
                       pltpu.SemaphoreType.DMA],
    )
    def sk(xw3, gc, sr, pf, zsm, out, idxc, idxr, xwb, pv, accw, idb,
           accs, sem):
        c = lax.axis_index("c")
        s = lax.axis_index("s")
        wid = s * NC + c
        rbase0 = wid * rows_per_w
        pltpu.sync_copy(pf, pv)
        zv = jnp.zeros((16,), jnp.float32)
        iot = lax.iota(jnp.int32, 16)
        for j in range(AROWS // CHUNK):
            for q in range(CHUNK // 16):
                idb[j, pl.ds(q * 16, 16)] = iot + (j * CHUNK + q * 16)

        @pl.loop(0, AROWS)
        def _z(r):
            for q in range(8):
                accw[r, pl.ds(q * 16, 16)] = zv

        @pl.when(s == 0)
        def _():
            pltpu.sync_copy(zsm.at[c], accs)
        plsc.subcore_barrier()

        @pl.loop(0, groups)
        def _grp(g):
            rbase = rbase0 + g * IG
            pltpu.sync_copy(gc.at[pl.ds(rbase, IG)], idxc)
            pltpu.sync_copy(sr.at[pl.ds(rbase, IG)], idxr)
            pltpu.sync_copy(xw3.at[pl.ds(rbase // (EB // CHUNK), nblk_g)],
                            xwb)
            ones = jnp.ones((16,), jnp.float32)
            for t in range(IG):
                for q in range(CHUNK // 16):
                    ic = idxc[t, pl.ds(q * 16, 16)] * 4
                    ir = idxr[t, pl.ds(q * 16, 16)] * SMW
                    xw = xwb[t // 4, t % 4, pl.ds(q * 16, 16)]
                    for k in range(3):
                        pc = plsc.load_gather(pv, [ic + k])
                        w = ir + k
                        plsc.addupdate_scatter(
                            accw, [lax.shift_right_logical(w, 7), w & 127],
                            xw * pc)
                    w = ir + 3
                    plsc.addupdate_scatter(
                        accw, [lax.shift_right_logical(w, 7), w & 127], xw)
                    w = ir + 4
                    plsc.addupdate_scatter(
                        accw, [lax.shift_right_logical(w, 7), w & 127], ones)

        for j in range(AROWS // CHUNK):
            pltpu.sync_copy(accw.at[pl.ds(j * CHUNK, CHUNK)],
                            accs.at[idb.at[j]], add=True)
        plsc.subcore_barrier()

        @pl.when(s == 0)
        def _():
            pltpu.sync_copy(accs, out.at[c])

    return sk


def kernel(x, pos, edge_index, We1, be1, We2, be2, Wx1, bx1, Wx2, bx2,
           Wh1, bh1, Wh2, bh2, Wfc, bfc):
    n, h_dim = x.shape
    num_layers = We1.shape[0]
    co = Wfc.shape[1]
    e = edge_index.shape[1]
    step = NW * IG * CHUNK
    e_pad = ((e + step - 1) // step) * step
    padn = e_pad - e

    i32 = jnp.int32
    row = edge_index[0].astype(i32)
    col = edge_index[1].astype(i32)
    g_row = jnp.concatenate([row, jnp.zeros((padn,), i32)]
                            ).reshape(e_pad // CHUNK, CHUNK)
    g_col = jnp.concatenate([col, jnp.zeros((padn,), i32)]
                            ).reshape(e_pad // CHUNK, CHUNK)
    s_row = jnp.concatenate([row, jnp.full((padn,), n, i32)]
                            ).reshape(e_pad // CHUNK, CHUNK)
    p8 = jnp.concatenate([pos, jnp.zeros((n, 5), jnp.float32)], axis=1)
    zeda = jnp.zeros((NC, n + 1, 128), jnp.float32)
    zsm = jnp.zeros((NC, AROWS, 128), jnp.float32)

    e_half = e_pad // 2
    hrows = e_half // CHUNK
    gr = [g_row[:hrows], g_row[hrows:]]
    gc = [g_col[:hrows], g_col[hrows:]]
    sr = [s_row[:hrows], s_row[hrows:]]

    tables = _tc_tables(n)
    edge_mlp = _tc_edge(e_half)
    node_mid = _tc_node_mid(n)
    node_fin = _tc_node_fin(n, co)
    dist2 = _sc_dist2(n, e_half)
    gather = _sc_gather(n, e_half)
    scatter_m = _sc_scatter_m(n, e_half)
    scatter_s = _sc_scatter_small(n, e_half)

    h = x
    ta, tb = tables(h, We1[0, 0:128], We1[0, 128:256])
    for l in range(num_layers):
        p_flat = p8[:, :4].reshape(-1)
        pay = [None, None]
        xw3 = [None, None]
        for hf in range(2):
            d2 = dist2(gr[hf], gc[hf], p_flat)
            ga, gb = gather(ta, tb, gr[hf], gc[hf])
            pay[hf], xw3[hf] = edge_mlp(
                ga, gb, d2, We1[l, 256:257], be1[l:l + 1],
                We2[l], be2[l:l + 1], Wx1[l], bx1[l:l + 1],
                Wx2[l], bx2[l:l + 1])
        accs = scatter_m(pay[0], sr[0], zeda)
        accs = scatter_m(pay[1], sr[1], accs)
        a0 = accs[0, :n]
        a1 = accs[1, :n]
        if l + 1 < num_layers:
            sm = scatter_s(xw3[0], gc[0], sr[0], p_flat, zsm)
            sm = scatter_s(xw3[1], gc[1], sr[1], p_flat, sm)
            sm = sm.reshape(NC, AROWS * 128)[:, :n * SMW].reshape(NC, n, SMW)
            h, p8, ta, tb = node_mid(h, p8, a0, a1, sm[0], sm[1],
                                     Wh1[l, 0:128], Wh1[l, 128:256],
                                     bh1[l:l + 1], Wh2[l], bh2[l:l + 1],
                                     We1[l + 1, 0:128], We1[l + 1, 128:256])
        else:
            y = node_fin(h, a0, a1, Wh1[l, 0:128], Wh1[l, 128:256],
                         bh1[l:l + 1], Wh2[l], bh2[l:l + 1], Wfc,
                         bfc.reshape(1, co))
    return y


# speedup vs baseline: 4.4759x; 1.0759x over previous
import functools

import jax
import jax.numpy as jnp
from jax import lax
from jax.experimental import pallas as pl
from jax.experimental.pallas import tpu as pltpu
from jax.experimental.pallas import tpu_sc as plsc

NC = 2
NS = 16
NW = NC * NS
CHUNK = 128
IG = 8
SB = 4
EB = 512
RB = 1000
SMW = 5


def _silu(v):
    return v * jax.nn.sigmoid(v)


def _tables_body(h_ref, wa_ref, wb_ref, ta_ref, tb_ref):
    h = h_ref[...]
    ta_ref[...] = jnp.dot(h, wa_ref[...], preferred_element_type=jnp.float32)
    tb_ref[...] = jnp.dot(h, wb_ref[...], preferred_element_type=jnp.float32)


def _edge_body(ga_ref, gb_ref, d2_ref, we1c_ref, be1_ref, w2_ref, b2_ref,
               wx1_ref, bx1_ref, wx2_ref, bx2_ref, m_ref, xw_ref):
    a = ga_ref[...]
    b = gb_ref[...]
    we1c = we1c_ref[...]
    outs = []
    for t in range(EB // CHUNK):
        d2row = d2_ref[0, t:t + 1, :]
        outs.append(lax.dot_general(
            d2row, we1c, (((0,), (0,)), ((), ())),
            preferred_element_type=jnp.float32))
    outer = jnp.concatenate(outs, axis=0)
    m1 = a + b + outer + be1_ref[...]
    t_ = _silu(m1)
    m = _silu(jnp.dot(t_, w2_ref[...], preferred_element_type=jnp.float32)
              + b2_ref[...])
    u = _silu(jnp.dot(m, wx1_ref[...], preferred_element_type=jnp.float32)
              + bx1_ref[...])
    m_ref[...] = m
    wx2 = wx2_ref[...]
    rows = []
    for t in range(EB // CHUNK):
        u_t = lax.slice(u, (t * CHUNK, 0), ((t + 1) * CHUNK, 128))
        rows.append(lax.dot_general(
            wx2, u_t, (((0,), (1,)), ((), ())),
            preferred_element_type=jnp.float32))
    xw = jnp.concatenate(rows, axis=0) + bx2_ref[0:1, 0:1]
    xw_ref[...] = xw[None]


def _node_mid_body(h_ref, p8_ref, a0_ref, a1_ref, s0_ref, s1_ref,
                   wh1h_ref, wh1a_ref, bh1_ref, wh2_ref, bh2_ref,
                   wa_ref, wb_ref, hn_ref, p8n_ref, ta_ref, tb_ref):
    h = h_ref[...]
    agg = a0_ref[...] + a1_ref[...]
    s = s0_ref[...] + s1_ref[...]
    pvec = s[:, 0:3]
    sxw = s[:, 3:4]
    cnt = jnp.maximum(s[:, 4:5], 1.0)
    p = p8_ref[...][:, :3]
    p_new = p + (p * sxw - pvec) / cnt
    u = _silu(jnp.dot(h, wh1h_ref[...], preferred_element_type=jnp.float32)
              + jnp.dot(agg, wh1a_ref[...], preferred_element_type=jnp.float32)
              + bh1_ref[...])
    h_new = h + jnp.dot(u, wh2_ref[...], preferred_element_type=jnp.float32) \
        + bh2_ref[...]
    n = h.shape[0]
    p8n = jnp.concatenate([p_new, jnp.zeros((n, 5), jnp.float32)], axis=1)
    hn_ref[...] = h_new
    p8n_ref[...] = p8n
    ta_ref[...] = jnp.dot(h_new, wa_ref[...],
                          preferred_element_type=jnp.float32)
    tb_ref[...] = jnp.dot(h_new, wb_ref[...],
                          preferred_element_type=jnp.float32)


def _node_fin_body(h_ref, a0_ref, a1_ref, wh1h_ref, wh1a_ref, bh1_ref,
                   wh2_ref, bh2_ref, wfc_ref, bfc_ref, y_ref):
    h = h_ref[...]
    agg = a0_ref[...] + a1_ref[...]
    u = _silu(jnp.dot(h, wh1h_ref[...], preferred_element_type=jnp.float32)
              + jnp.dot(agg, wh1a_ref[...], preferred_element_type=jnp.float32)
              + bh1_ref[...])
    h_new = h + jnp.dot(u, wh2_ref[...], preferred_element_type=jnp.float32) \
        + bh2_ref[...]
    y_ref[...] = (jnp.dot(h_new, wfc_ref[...],
                          preferred_element_type=jnp.float32) + bfc_ref[...])


def _wspec(r, c):
    return pl.BlockSpec((r, c), lambda i: (0, 0))


def _tc_tables(n):
    bs = pl.BlockSpec((RB, 128), lambda i: (i, 0))
    return pl.pallas_call(
        _tables_body,
        grid=(n // RB,),
        in_specs=[bs, _wspec(128, 128), _wspec(128, 128)],
        out_specs=[bs, bs],
        out_shape=[jax.ShapeDtypeStruct((n, 128), jnp.float32)] * 2,
    )


def _tc_edge(e_pad):
    bw = pl.BlockSpec((EB, 128), lambda i: (i, 0))
    bx = pl.BlockSpec((1, EB // CHUNK, CHUNK), lambda i: (i, 0, 0))
    return pl.pallas_call(
        _edge_body,
        grid=(e_pad // EB,),
        in_specs=[bw, bw, bx, _wspec(1, 128), _wspec(1, 128),
                  _wspec(128, 128), _wspec(1, 128), _wspec(128, 128),
                  _wspec(1, 128), _wspec(128, 1), _wspec(1, 128)],
        out_specs=[bw, bx],
        out_shape=[jax.ShapeDtypeStruct((e_pad, 128), jnp.float32),
                   jax.ShapeDtypeStruct((e_pad // EB, EB // CHUNK, CHUNK),
                                        jnp.float32)],
    )


def _tc_node_mid(n):
    bh = pl.BlockSpec((RB, 128), lambda i: (i, 0))
    bp = pl.BlockSpec((RB, 8), lambda i: (i, 0))
    bs = pl.BlockSpec((RB, SMW), lambda i: (i, 0))
    return pl.pallas_call(
        _node_mid_body,
        grid=(n // RB,),
        in_specs=[bh, bp, bh, bh, bs, bs, _wspec(128, 128), _wspec(128, 128),
                  _wspec(1, 128), _wspec(128, 128), _wspec(1, 128),
                  _wspec(128, 128), _wspec(128, 128)],
        out_specs=[bh, bp, bh, bh],
        out_shape=[jax.ShapeDtypeStruct((n, 128), jnp.float32),
                   jax.ShapeDtypeStruct((n, 8), jnp.float32),
                   jax.ShapeDtypeStruct((n, 128), jnp.float32),
                   jax.ShapeDtypeStruct((n, 128), jnp.float32)],
    )


def _tc_node_fin(n, co):
    bh = pl.BlockSpec((RB, 128), lambda i: (i, 0))
    return pl.pallas_call(
        _node_fin_body,
        grid=(n // RB,),
        in_specs=[bh, bh, bh, _wspec(128, 128), _wspec(128, 128),
                  _wspec(1, 128), _wspec(128, 128), _wspec(1, 128),
                  _wspec(128, co), _wspec(1, co)],
        out_specs=pl.BlockSpec((RB, co), lambda i: (i, 0)),
        out_shape=jax.ShapeDtypeStruct((n, co), jnp.float32),
    )


def _mesh():
    return plsc.VectorSubcoreMesh(core_axis_name="c", subcore_axis_name="s")


_SC_PARAMS = pltpu.CompilerParams(needs_layout_passes=False)


def _sc_dist2(n, e_pad):
    rows = e_pad // CHUNK
    rows_per_w = rows // NW
    d_groups = rows_per_w // IG
    nblk = e_pad // EB

    @functools.partial(
        pl.kernel,
        out_type=jax.ShapeDtypeStruct((nblk, EB // CHUNK, CHUNK),
                                      jnp.float32),
        mesh=_mesh(),
        compiler_params=_SC_PARAMS,
        scratch_types=[pltpu.VMEM((IG, CHUNK), jnp.int32),
                       pltpu.VMEM((IG, CHUNK), jnp.int32),
                       pltpu.VMEM((4 * n,), jnp.float32),
                       pltpu.VMEM((IG // (EB // CHUNK), EB // CHUNK, CHUNK),
                                  jnp.float32)],
    )
    def dk(gr, gc, pf, d2, idxr, idxc, pv, dbuf):
        wid = lax.axis_index("s") * NC + lax.axis_index("c")
        rbase0 = wid * rows_per_w
        pltpu.sync_copy(pf, pv)

        @pl.loop(0, d_groups)
        def _dgrp(g):
            rbase = rbase0 + g * IG
            pltpu.sync_copy(gr.at[pl.ds(rbase, IG)], idxr)
            pltpu.sync_copy(gc.at[pl.ds(rbase, IG)], idxc)
            for t in range(IG):
                for q in range(CHUNK // 16):
                    ir = idxr[t, pl.ds(q * 16, 16)] * 4
                    ic = idxc[t, pl.ds(q * 16, 16)] * 4
                    d2v = jnp.zeros((16,), jnp.float32)
                    for k in range(3):
                        dr = plsc.load_gather(pv, [ir + k])
                        dc = plsc.load_gather(pv, [ic + k])
                        dk_ = dr - dc
                        d2v = d2v + dk_ * dk_
                    dbuf[t // 4, t % 4, pl.ds(q * 16, 16)] = d2v
            pltpu.sync_copy(dbuf, d2.at[pl.ds(rbase // (EB // CHUNK),
                                              IG // (EB // CHUNK))])

    return dk


GSB = 2


def _sc_gather(n, e_pad):
    rows = e_pad // CHUNK
    rows_per_s = rows // NS
    g_groups = rows_per_s // IG

    @functools.partial(
        pl.kernel,
        out_type=(jax.ShapeDtypeStruct((e_pad, 128), jnp.float32),
                  jax.ShapeDtypeStruct((e_pad, 128), jnp.float32)),
        mesh=_mesh(),
        compiler_params=_SC_PARAMS,
        scratch_types=[pltpu.VMEM((IG, CHUNK), jnp.int32),
                       pltpu.VMEM((GSB * CHUNK, 128), jnp.float32),
                       pltpu.VMEM_SHARED((n, 128), jnp.float32),
                       pltpu.SemaphoreType.DMA],
    )
    def gk(ta, tb, gr, gc, ga, gb, idx, buf, tab, sem):
        c = lax.axis_index("c")
        s = lax.axis_index("s")

        @pl.when(jnp.logical_and(s == 0, c == 0))
        def _():
            pltpu.sync_copy(ta, tab)

        @pl.when(jnp.logical_and(s == 0, c == 1))
        def _():
            pltpu.sync_copy(tb, tab)
        plsc.subcore_barrier()

        gbase0 = s * rows_per_s

        @pl.loop(0, g_groups)
        def _ggrp(g):
            rbase = gbase0 + g * IG

            @pl.when(c == 0)
            def _():
                pltpu.sync_copy(gr.at[pl.ds(rbase, IG)], idx)

            @pl.when(c == 1)
            def _():
                pltpu.sync_copy(gc.at[pl.ds(rbase, IG)], idx)
            for half in range(IG // GSB):
                ebase = (rbase + half * GSB) * CHUNK
                descs = [pltpu.async_copy(
                    tab.at[idx.at[half * GSB + t]],
                    buf.at[pl.ds(t * CHUNK, CHUNK)], sem)
                    for t in range(GSB)]
                for d in descs:
                    d.wait()

                @pl.when(c == 0)
                def _():
                    pltpu.sync_copy(buf, ga.at[pl.ds(ebase, GSB * CHUNK)])

                @pl.when(c == 1)
                def _():
                    pltpu.sync_copy(buf, gb.at[pl.ds(ebase, GSB * CHUNK)])

    return gk


def _sc_scatter_m(n, e_pad):
    rows = e_pad // CHUNK
    rows_per_w = rows // NW
    groups = rows_per_w // IG
    SBM = 2

    @functools.partial(
        pl.kernel,
        out_type=jax.ShapeDtypeStruct((NC, n + 1, 128), jnp.float32),
        mesh=_mesh(),
        compiler_params=_SC_PARAMS,
        scratch_types=[pltpu.VMEM((IG, CHUNK), jnp.int32),
                       pltpu.VMEM((SBM * CHUNK, 128), jnp.float32),
                       pltpu.VMEM_SHARED((n + 1, 128), jnp.float32),
                       pltpu.SemaphoreType.DMA],
    )
    def sk(pay, sr, zed, out, idxv, buf, acc, sem):
        c = lax.axis_index("c")
        s = lax.axis_index("s")
        wid = s * NC + c
        rbase0 = wid * rows_per_w

        @pl.when(s == 0)
        def _():
            pltpu.sync_copy(zed.at[c], acc)
        plsc.subcore_barrier()

        @pl.loop(0, groups)
        def _grp(g):
            rbase = rbase0 + g * IG
            pltpu.sync_copy(sr.at[pl.ds(rbase, IG)], idxv)
            for half in range(IG // SBM):
                ebase = (rbase + half * SBM) * CHUNK
                pltpu.sync_copy(pay.at[pl.ds(ebase, SBM * CHUNK)], buf)
                for t in range(SBM):
                    pltpu.sync_copy(buf.at[pl.ds(t * CHUNK, CHUNK)],
                                    acc.at[idxv.at[half * SBM + t]], add=True)

        plsc.subcore_barrier()

        @pl.when(s == 0)
        def _():
            pltpu.sync_copy(acc, out.at[c])

    return sk


AROWS = 512


def _sc_scatter_small(n, e_pad):
    rows = e_pad // CHUNK
    rows_per_w = rows // NW
    groups = rows_per_w // IG
    nblk_g = IG // (EB // CHUNK)

    @functools.partial(
        pl.kernel,
        out_type=jax.ShapeDtypeStruct((NC, AROWS, 128), jnp.float32),
        mesh=_mesh(),
        compiler_params=_SC_PARAMS,
        scratch_types=[pltpu.VMEM((IG, CHUNK), jnp.int32),
                       pltpu.VMEM((IG, CHUNK), jnp.int32),
                       pltpu.VMEM((nblk_g, EB // CHUNK, CHUNK), jnp.float32),
                       pltpu.VMEM((4 * n,), jnp.float32),
                       pltpu.VMEM((AROWS, 128), jnp.float32),
                       pltpu.VMEM((AROWS // CHUNK, CHUNK), jnp.int32),
                       pltpu.VMEM_SHARED((AROWS, 128), jnp.float32),
                       pltpu.SemaphoreType.DMA],
    )
    def sk(xw3, gc, sr, pf, zsm, out, idxc, idxr, xwb, pv, accw, idb,
           accs, sem):
        c = lax.axis_index("c")
        s = lax.axis_index("s")
        wid = s * NC + c
        rbase0 = wid * rows_per_w
        pltpu.sync_copy(pf, pv)
        zv = jnp.zeros((16,), jnp.float32)
        iot = lax.iota(jnp.int32, 16)
        for j in range(AROWS // CHUNK):
            for q in range(CHUNK // 16):
                idb[j, pl.ds(q * 16, 16)] = iot + (j * CHUNK + q * 16)

        @pl.loop(0, AROWS)
        def _z(r):
            for q in range(8):
                accw[r, pl.ds(q * 16, 16)] = zv

        @pl.when(s == 0)
        def _():
            pltpu.sync_copy(zsm.at[c], accs)
        plsc.subcore_barrier()

        @pl.loop(0, groups)
        def _grp(g):
            rbase = rbase0 + g * IG
            pltpu.sync_copy(gc.at[pl.ds(rbase, IG)], idxc)
            pltpu.sync_copy(sr.at[pl.ds(rbase, IG)], idxr)
            pltpu.sync_copy(xw3.at[pl.ds(rbase // (EB // CHUNK), nblk_g)],
                            xwb)
            ones = jnp.ones((16,), jnp.float32)
            for t in range(IG):
                for q in range(CHUNK // 16):
                    ic = idxc[t, pl.ds(q * 16, 16)] * 4
                    ir = idxr[t, pl.ds(q * 16, 16)] * SMW
                    xw = xwb[t // 4, t % 4, pl.ds(q * 16, 16)]
                    for k in range(3):
                        pc = plsc.load_gather(pv, [ic + k])
                        w = ir + k
                        plsc.addupdate_scatter(
                            accw, [lax.shift_right_logical(w, 7), w & 127],
                            xw * pc)
                    w = ir + 3
                    plsc.addupdate_scatter(
                        accw, [lax.shift_right_logical(w, 7), w & 127], xw)
                    w = ir + 4
                    plsc.addupdate_scatter(
                        accw, [lax.shift_right_logical(w, 7), w & 127], ones)

        for j in range(AROWS // CHUNK):
            pltpu.sync_copy(accw.at[pl.ds(j * CHUNK, CHUNK)],
                            accs.at[idb.at[j]], add=True)
        plsc.subcore_barrier()

        @pl.when(s == 0)
        def _():
            pltpu.sync_copy(accs, out.at[c])

    return sk


def kernel(x, pos, edge_index, We1, be1, We2, be2, Wx1, bx1, Wx2, bx2,
           Wh1, bh1, Wh2, bh2, Wfc, bfc):
    n, h_dim = x.shape
    num_layers = We1.shape[0]
    co = Wfc.shape[1]
    e = edge_index.shape[1]
    step = NW * IG * CHUNK
    e_pad = ((e + step - 1) // step) * step
    padn = e_pad - e

    i32 = jnp.int32
    row = edge_index[0].astype(i32)
    col = edge_index[1].astype(i32)
    g_row = jnp.concatenate([row, jnp.zeros((padn,), i32)]
                            ).reshape(e_pad // CHUNK, CHUNK)
    g_col = jnp.concatenate([col, jnp.zeros((padn,), i32)]
                            ).reshape(e_pad // CHUNK, CHUNK)
    s_row = jnp.concatenate([row, jnp.full((padn,), n, i32)]
                            ).reshape(e_pad // CHUNK, CHUNK)
    p8 = jnp.concatenate([pos, jnp.zeros((n, 5), jnp.float32)], axis=1)
    zeda = jnp.zeros((NC, n + 1, 128), jnp.float32)
    zsm = jnp.zeros((NC, AROWS, 128), jnp.float32)

    nchunk = e_pad // step
    while nchunk > 1 and (nchunk > 5 or e_pad % (nchunk * step)):
        nchunk -= 1
    e_ch = e_pad // nchunk
    crows = e_ch // CHUNK
    gr = [g_row[i * crows:(i + 1) * crows] for i in range(nchunk)]
    gc = [g_col[i * crows:(i + 1) * crows] for i in range(nchunk)]
    sr = [s_row[i * crows:(i + 1) * crows] for i in range(nchunk)]

    tables = _tc_tables(n)
    edge_mlp = _tc_edge(e_ch)
    node_mid = _tc_node_mid(n)
    node_fin = _tc_node_fin(n, co)
    dist2 = _sc_dist2(n, e_ch)
    gather = _sc_gather(n, e_ch)
    scatter_m = _sc_scatter_m(n, e_ch)
    scatter_s = _sc_scatter_small(n, e_ch)

    h = x
    ta, tb = tables(h, We1[0, 0:128], We1[0, 128:256])
    for l in range(num_layers):
        p_flat = p8[:, :4].reshape(-1)
        pay = [None] * nchunk
        xw3 = [None] * nchunk
        for hf in range(nchunk):
            d2 = dist2(gr[hf], gc[hf], p_flat)
            ga, gb = gather(ta, tb, gr[hf], gc[hf])
            pay[hf], xw3[hf] = edge_mlp(
                ga, gb, d2, We1[l, 256:257], be1[l:l + 1],
                We2[l], be2[l:l + 1], Wx1[l], bx1[l:l + 1],
                Wx2[l], bx2[l:l + 1])
        accs = zeda
        for hf in range(nchunk):
            accs = scatter_m(pay[hf], sr[hf], accs)
        a0 = accs[0, :n]
        a1 = accs[1, :n]
        if l + 1 < num_layers:
            sm = zsm
            for hf in range(nchunk):
                sm = scatter_s(xw3[hf], gc[hf], sr[hf], p_flat, sm)
            sm = sm.reshape(NC, AROWS * 128)[:, :n * SMW].reshape(NC, n, SMW)
            h, p8, ta, tb = node_mid(h, p8, a0, a1, sm[0], sm[1],
                                     Wh1[l, 0:128], Wh1[l, 128:256],
                                     bh1[l:l + 1], Wh2[l], bh2[l:l + 1],
                                     We1[l + 1, 0:128], We1[l + 1, 128:256])
        else:
            y = node_fin(h, a0, a1, Wh1[l, 0:128], Wh1[l, 128:256],
                         bh1[l:l + 1], Wh2[l], bh2[l:l + 1], Wfc,
                         bfc.reshape(1, co))
    return y


# speedup vs baseline: 4.5533x; 1.0173x over previous
import functools

import jax
import jax.numpy as jnp
from jax import lax
from jax.experimental import pallas as pl
from jax.experimental.pallas import tpu as pltpu
from jax.experimental.pallas import tpu_sc as plsc

NC = 2
NS = 16
NW = NC * NS
CHUNK = 128
IG = 8
SB = 4
EB = 512
RB = 1000
SMW = 5


def _silu(v):
    return v * jax.nn.sigmoid(v)


def _tables_body(h_ref, wa_ref, wb_ref, ta_ref, tb_ref):
    h = h_ref[...]
    ta_ref[...] = jnp.dot(h, wa_ref[...], preferred_element_type=jnp.float32)
    tb_ref[...] = jnp.dot(h, wb_ref[...], preferred_element_type=jnp.float32)


def _edge_body(ga_ref, gb_ref, d2_ref, we1c_ref, be1_ref, w2_ref, b2_ref,
               wx1_ref, bx1_ref, wx2_ref, bx2_ref, m_ref, xw_ref):
    a = ga_ref[...]
    b = gb_ref[...]
    we1c = we1c_ref[...]
    outs = []
    for t in range(EB // CHUNK):
        d2row = d2_ref[0, t:t + 1, :]
        outs.append(lax.dot_general(
            d2row, we1c, (((0,), (0,)), ((), ())),
            preferred_element_type=jnp.float32))
    outer = jnp.concatenate(outs, axis=0)
    m1 = a + b + outer + be1_ref[...]
    t_ = _silu(m1)
    m = _silu(jnp.dot(t_, w2_ref[...], preferred_element_type=jnp.float32)
              + b2_ref[...])
    u = _silu(jnp.dot(m, wx1_ref[...], preferred_element_type=jnp.float32)
              + bx1_ref[...])
    m_ref[...] = m
    wx2 = wx2_ref[...]
    rows = []
    for t in range(EB // CHUNK):
        u_t = lax.slice(u, (t * CHUNK, 0), ((t + 1) * CHUNK, 128))
        rows.append(lax.dot_general(
            wx2, u_t, (((0,), (1,)), ((), ())),
            preferred_element_type=jnp.float32))
    xw = jnp.concatenate(rows, axis=0) + bx2_ref[0:1, 0:1]
    xw_ref[...] = xw[None]


def _node_mid_body(h_ref, p8_ref, a0_ref, a1_ref, s0_ref, s1_ref,
                   wh1h_ref, wh1a_ref, bh1_ref, wh2_ref, bh2_ref,
                   wa_ref, wb_ref, hn_ref, p8n_ref, ta_ref, tb_ref):
    h = h_ref[...]
    agg = a0_ref[...] + a1_ref[...]
    s = s0_ref[...] + s1_ref[...]
    pvec = s[:, 0:3]
    sxw = s[:, 3:4]
    cnt = jnp.maximum(s[:, 4:5], 1.0)
    p = p8_ref[...][:, :3]
    p_new = p + (p * sxw - pvec) / cnt
    u = _silu(jnp.dot(h, wh1h_ref[...], preferred_element_type=jnp.float32)
              + jnp.dot(agg, wh1a_ref[...], preferred_element_type=jnp.float32)
              + bh1_ref[...])
    h_new = h + jnp.dot(u, wh2_ref[...], preferred_element_type=jnp.float32) \
        + bh2_ref[...]
    n = h.shape[0]
    p8n = jnp.concatenate([p_new, jnp.zeros((n, 5), jnp.float32)], axis=1)
    hn_ref[...] = h_new
    p8n_ref[...] = p8n
    ta_ref[...] = jnp.dot(h_new, wa_ref[...],
                          preferred_element_type=jnp.float32)
    tb_ref[...] = jnp.dot(h_new, wb_ref[...],
                          preferred_element_type=jnp.float32)


def _node_fin_body(h_ref, a0_ref, a1_ref, wh1h_ref, wh1a_ref, bh1_ref,
                   wh2_ref, bh2_ref, wfc_ref, bfc_ref, y_ref):
    h = h_ref[...]
    agg = a0_ref[...] + a1_ref[...]
    u = _silu(jnp.dot(h, wh1h_ref[...], preferred_element_type=jnp.float32)
              + jnp.dot(agg, wh1a_ref[...], preferred_element_type=jnp.float32)
              + bh1_ref[...])
    h_new = h + jnp.dot(u, wh2_ref[...], preferred_element_type=jnp.float32) \
        + bh2_ref[...]
    y_ref[...] = (jnp.dot(h_new, wfc_ref[...],
                          preferred_element_type=jnp.float32) + bfc_ref[...])


def _wspec(r, c):
    return pl.BlockSpec((r, c), lambda i: (0, 0))


def _tc_tables(n):
    bs = pl.BlockSpec((RB, 128), lambda i: (i, 0))
    return pl.pallas_call(
        _tables_body,
        grid=(n // RB,),
        in_specs=[bs, _wspec(128, 128), _wspec(128, 128)],
        out_specs=[bs, bs],
        out_shape=[jax.ShapeDtypeStruct((n, 128), jnp.float32)] * 2,
    )


def _tc_edge(e_pad):
    bw = pl.BlockSpec((EB, 128), lambda i: (i, 0))
    bx = pl.BlockSpec((1, EB // CHUNK, CHUNK), lambda i: (i, 0, 0))
    return pl.pallas_call(
        _edge_body,
        grid=(e_pad // EB,),
        in_specs=[bw, bw, bx, _wspec(1, 128), _wspec(1, 128),
                  _wspec(128, 128), _wspec(1, 128), _wspec(128, 128),
                  _wspec(1, 128), _wspec(128, 1), _wspec(1, 128)],
        out_specs=[bw, bx],
        out_shape=[jax.ShapeDtypeStruct((e_pad, 128), jnp.float32),
                   jax.ShapeDtypeStruct((e_pad // EB, EB // CHUNK, CHUNK),
                                        jnp.float32)],
    )


def _tc_node_mid(n):
    bh = pl.BlockSpec((RB, 128), lambda i: (i, 0))
    bp = pl.BlockSpec((RB, 8), lambda i: (i, 0))
    bs = pl.BlockSpec((RB, SMW), lambda i: (i, 0))
    return pl.pallas_call(
        _node_mid_body,
        grid=(n // RB,),
        in_specs=[bh, bp, bh, bh, bs, bs, _wspec(128, 128), _wspec(128, 128),
                  _wspec(1, 128), _wspec(128, 128), _wspec(1, 128),
                  _wspec(128, 128), _wspec(128, 128)],
        out_specs=[bh, bp, bh, bh],
        out_shape=[jax.ShapeDtypeStruct((n, 128), jnp.float32),
                   jax.ShapeDtypeStruct((n, 8), jnp.float32),
                   jax.ShapeDtypeStruct((n, 128), jnp.float32),
                   jax.ShapeDtypeStruct((n, 128), jnp.float32)],
    )


def _tc_node_fin(n, co):
    bh = pl.BlockSpec((RB, 128), lambda i: (i, 0))
    return pl.pallas_call(
        _node_fin_body,
        grid=(n // RB,),
        in_specs=[bh, bh, bh, _wspec(128, 128), _wspec(128, 128),
                  _wspec(1, 128), _wspec(128, 128), _wspec(1, 128),
                  _wspec(128, co), _wspec(1, co)],
        out_specs=pl.BlockSpec((RB, co), lambda i: (i, 0)),
        out_shape=jax.ShapeDtypeStruct((n, co), jnp.float32),
    )


def _mesh():
    return plsc.VectorSubcoreMesh(core_axis_name="c", subcore_axis_name="s")


_SC_PARAMS = pltpu.CompilerParams(needs_layout_passes=False)


GSB = 1


def _sc_gather(n, e_pad):
    rows = e_pad // CHUNK
    rows_per_s = rows // NS
    g_groups = rows_per_s // IG
    nblk = e_pad // EB

    @functools.partial(
        pl.kernel,
        out_type=(jax.ShapeDtypeStruct((e_pad, 128), jnp.float32),
                  jax.ShapeDtypeStruct((e_pad, 128), jnp.float32),
                  jax.ShapeDtypeStruct((nblk, EB // CHUNK, CHUNK),
                                       jnp.float32)),
        mesh=_mesh(),
        compiler_params=_SC_PARAMS,
        scratch_types=[pltpu.VMEM((IG, CHUNK), jnp.int32),
                       pltpu.VMEM((IG, CHUNK), jnp.int32),
                       pltpu.VMEM((GSB * CHUNK, 128), jnp.float32),
                       pltpu.VMEM((3 * n,), jnp.float32),
                       pltpu.VMEM((1, EB // CHUNK, CHUNK), jnp.float32),
                       pltpu.VMEM_SHARED((n, 128), jnp.float32),
                       pltpu.SemaphoreType.DMA],
    )
    def gk(ta, tb, gr, gc, pf, ga, gb, d2, idx, idx2, buf, pv, dbuf, tab,
           sem):
        c = lax.axis_index("c")
        s = lax.axis_index("s")

        @pl.when(jnp.logical_and(s == 0, c == 0))
        def _():
            pltpu.sync_copy(ta, tab)

        @pl.when(jnp.logical_and(s == 0, c == 1))
        def _():
            pltpu.sync_copy(tb, tab)

        pltpu.sync_copy(pf, pv)
        plsc.subcore_barrier()

        gbase0 = s * rows_per_s

        def _d2rows(ir_ref, ic_ref, ro):
            for t in range(IG // 2):
                for q in range(CHUNK // 16):
                    ir = ir_ref[ro + t, pl.ds(q * 16, 16)] * 3
                    ic = ic_ref[ro + t, pl.ds(q * 16, 16)] * 3
                    d2v = jnp.zeros((16,), jnp.float32)
                    for k in range(3):
                        dr = plsc.load_gather(pv, [ir + k])
                        dc = plsc.load_gather(pv, [ic + k])
                        dk_ = dr - dc
                        d2v = d2v + dk_ * dk_
                    dbuf[0, t, pl.ds(q * 16, 16)] = d2v

        @pl.loop(0, g_groups)
        def _ggrp(g):
            rbase = gbase0 + g * IG

            @pl.when(c == 0)
            def _():
                pltpu.sync_copy(gr.at[pl.ds(rbase, IG)], idx)
                pltpu.sync_copy(gc.at[pl.ds(rbase, IG)], idx2)

            @pl.when(c == 1)
            def _():
                pltpu.sync_copy(gc.at[pl.ds(rbase, IG)], idx)
                pltpu.sync_copy(gr.at[pl.ds(rbase, IG)], idx2)
            @pl.when(c == 0)
            def _():
                _d2rows(idx, idx2, 0)

            @pl.when(c == 1)
            def _():
                _d2rows(idx2, idx, IG // 2)
            pltpu.sync_copy(
                dbuf, d2.at[pl.ds(rbase // (EB // CHUNK) + c, 1)])
            for half in range(IG // GSB):
                ebase = (rbase + half * GSB) * CHUNK
                for t in range(GSB):
                    pltpu.sync_copy(tab.at[idx.at[half * GSB + t]],
                                    buf.at[pl.ds(t * CHUNK, CHUNK)])

                @pl.when(c == 0)
                def _():
                    pltpu.sync_copy(buf, ga.at[pl.ds(ebase, GSB * CHUNK)])

                @pl.when(c == 1)
                def _():
                    pltpu.sync_copy(buf, gb.at[pl.ds(ebase, GSB * CHUNK)])

    return gk


def _sc_scatter_m(n, e_pad):
    rows = e_pad // CHUNK
    rows_per_w = rows // NW
    groups = rows_per_w // IG
    SBM = 2

    @functools.partial(
        pl.kernel,
        out_type=jax.ShapeDtypeStruct((NC, n + 1, 128), jnp.float32),
        mesh=_mesh(),
        compiler_params=_SC_PARAMS,
        scratch_types=[pltpu.VMEM((IG, CHUNK), jnp.int32),
                       pltpu.VMEM((SBM * CHUNK, 128), jnp.float32),
                       pltpu.VMEM_SHARED((n + 1, 128), jnp.float32),
                       pltpu.SemaphoreType.DMA],
    )
    def sk(pay, sr, zed, out, idxv, buf, acc, sem):
        c = lax.axis_index("c")
        s = lax.axis_index("s")
        wid = s * NC + c
        rbase0 = wid * rows_per_w

        @pl.when(s == 0)
        def _():
            pltpu.sync_copy(zed.at[c], acc)
        plsc.subcore_barrier()

        @pl.loop(0, groups)
        def _grp(g):
            rbase = rbase0 + g * IG
            pltpu.sync_copy(sr.at[pl.ds(rbase, IG)], idxv)
            for half in range(IG // SBM):
                ebase = (rbase + half * SBM) * CHUNK
                pltpu.sync_copy(pay.at[pl.ds(ebase, SBM * CHUNK)], buf)
                for t in range(SBM):
                    pltpu.sync_copy(buf.at[pl.ds(t * CHUNK, CHUNK)],
                                    acc.at[idxv.at[half * SBM + t]], add=True)

        plsc.subcore_barrier()

        @pl.when(s == 0)
        def _():
            pltpu.sync_copy(acc, out.at[c])

    return sk


AROWS = 512


def _sc_scatter_small(n, e_pad):
    rows = e_pad // CHUNK
    rows_per_w = rows // NW
    groups = rows_per_w // IG
    nblk_g = IG // (EB // CHUNK)

    @functools.partial(
        pl.kernel,
        out_type=jax.ShapeDtypeStruct((NC, AROWS, 128), jnp.float32),
        mesh=_mesh(),
        compiler_params=_SC_PARAMS,
        scratch_types=[pltpu.VMEM((IG, CHUNK), jnp.int32),
                       pltpu.VMEM((IG, CHUNK), jnp.int32),
                       pltpu.VMEM((nblk_g, EB // CHUNK, CHUNK), jnp.float32),
                       pltpu.VMEM((4 * n,), jnp.float32),
                       pltpu.VMEM((AROWS, 128), jnp.float32),
                       pltpu.VMEM((AROWS // CHUNK, CHUNK), jnp.int32),
                       pltpu.VMEM_SHARED((AROWS, 128), jnp.float32),
                       pltpu.SemaphoreType.DMA],
    )
    def sk(xw3, gc, sr, pf, zsm, out, idxc, idxr, xwb, pv, accw, idb,
           accs, sem):
        c = lax.axis_index("c")
        s = lax.axis_index("s")
        wid = s * NC + c
        rbase0 = wid * rows_per_w
        pltpu.sync_copy(pf, pv)
        zv = jnp.zeros((16,), jnp.float32)
        iot = lax.iota(jnp.int32, 16)
        for j in range(AROWS // CHUNK):
            for q in range(CHUNK // 16):
                idb[j, pl.ds(q * 16, 16)] = iot + (j * CHUNK + q * 16)

        @pl.loop(0, AROWS)
        def _z(r):
            for q in range(8):
                accw[r, pl.ds(q * 16, 16)] = zv

        @pl.when(s == 0)
        def _():
            pltpu.sync_copy(zsm.at[c], accs)
        plsc.subcore_barrier()

        @pl.loop(0, groups)
        def _grp(g):
            rbase = rbase0 + g * IG
            pltpu.sync_copy(gc.at[pl.ds(rbase, IG)], idxc)
            pltpu.sync_copy(sr.at[pl.ds(rbase, IG)], idxr)
            pltpu.sync_copy(xw3.at[pl.ds(rbase // (EB // CHUNK), nblk_g)],
                            xwb)
            ones = jnp.ones((16,), jnp.float32)
            for t in range(IG):
                for q in range(CHUNK // 16):
                    ic = idxc[t, pl.ds(q * 16, 16)] * 4
                    ir = idxr[t, pl.ds(q * 16, 16)] * SMW
                    xw = xwb[t // 4, t % 4, pl.ds(q * 16, 16)]
                    for k in range(3):
                        pc = plsc.load_gather(pv, [ic + k])
                        w = ir + k
                        plsc.addupdate_scatter(
                            accw, [lax.shift_right_logical(w, 7), w & 127],
                            xw * pc)
                    w = ir + 3
                    plsc.addupdate_scatter(
                        accw, [lax.shift_right_logical(w, 7), w & 127], xw)
                    w = ir + 4
                    plsc.addupdate_scatter(
                        accw, [lax.shift_right_logical(w, 7), w & 127], ones)

        for j in range(AROWS // CHUNK):
            pltpu.sync_copy(accw.at[pl.ds(j * CHUNK, CHUNK)],
                            accs.at[idb.at[j]], add=True)
        plsc.subcore_barrier()

        @pl.when(s == 0)
        def _():
            pltpu.sync_copy(accs, out.at[c])

    return sk


def kernel(x, pos, edge_index, We1, be1, We2, be2, Wx1, bx1, Wx2, bx2,
           Wh1, bh1, Wh2, bh2, Wfc, bfc):
    n, h_dim = x.shape
    num_layers = We1.shape[0]
    co = Wfc.shape[1]
    e = edge_index.shape[1]
    step = NW * IG * CHUNK
    e_pad = ((e + step - 1) // step) * step
    padn = e_pad - e

    i32 = jnp.int32
    row = edge_index[0].astype(i32)
    col = edge_index[1].astype(i32)
    g_row = jnp.concatenate([row, jnp.zeros((padn,), i32)]
                            ).reshape(e_pad // CHUNK, CHUNK)
    g_col = jnp.concatenate([col, jnp.zeros((padn,), i32)]
                            ).reshape(e_pad // CHUNK, CHUNK)
    s_row = jnp.concatenate([row, jnp.full((padn,), n, i32)]
                            ).reshape(e_pad // CHUNK, CHUNK)
    p8 = jnp.concatenate([pos, jnp.zeros((n, 5), jnp.float32)], axis=1)
    zeda = jnp.zeros((NC, n + 1, 128), jnp.float32)
    zsm = jnp.zeros((NC, AROWS, 128), jnp.float32)

    nchunk = e_pad // step
    while nchunk > 1 and (nchunk > 5 or e_pad % (nchunk * step)):
        nchunk -= 1
    e_ch = e_pad // nchunk
    crows = e_ch // CHUNK
    gr = [g_row[i * crows:(i + 1) * crows] for i in range(nchunk)]
    gc = [g_col[i * crows:(i + 1) * crows] for i in range(nchunk)]
    sr = [s_row[i * crows:(i + 1) * crows] for i in range(nchunk)]

    tables = _tc_tables(n)
    edge_mlp = _tc_edge(e_ch)
    node_mid = _tc_node_mid(n)
    node_fin = _tc_node_fin(n, co)
    gather = _sc_gather(n, e_ch)
    scatter_m = _sc_scatter_m(n, e_ch)
    scatter_s = _sc_scatter_small(n, e_ch)

    h = x
    ta, tb = tables(h, We1[0, 0:128], We1[0, 128:256])
    for l in range(num_layers):
        p_flat = p8[:, :4].reshape(-1)
        p_flat3 = p8[:, :3].reshape(-1)
        pay = [None] * nchunk
        xw3 = [None] * nchunk
        for hf in range(nchunk):
            ga, gb, d2 = gather(ta, tb, gr[hf], gc[hf], p_flat3)
            pay[hf], xw3[hf] = edge_mlp(
                ga, gb, d2, We1[l, 256:257], be1[l:l + 1],
                We2[l], be2[l:l + 1], Wx1[l], bx1[l:l + 1],
                Wx2[l], bx2[l:l + 1])
        accs = zeda
        for hf in range(nchunk):
            accs = scatter_m(pay[hf], sr[hf], accs)
        a0 = accs[0, :n]
        a1 = accs[1, :n]
        if l + 1 < num_layers:
            sm = zsm
            for hf in range(nchunk):
                sm = scatter_s(xw3[hf], gc[hf], sr[hf], p_flat, sm)
            sm = sm.reshape(NC, AROWS * 128)[:, :n * SMW].reshape(NC, n, SMW)
            h, p8, ta, tb = node_mid(h, p8, a0, a1, sm[0], sm[1],
                                     Wh1[l, 0:128], Wh1[l, 128:256],
                                     bh1[l:l + 1], Wh2[l], bh2[l:l + 1],
                                     We1[l + 1, 0:128], We1[l + 1, 128:256])
        else:
            y = node_fin(h, a0, a1, Wh1[l, 0:128], Wh1[l, 128:256],
                         bh1[l:l + 1], Wh2[l], bh2[l:l + 1], Wfc,
                         bfc.reshape(1, co))
    return y
